# baseline (speedup 1.0000x reference)
import jax
import jax.numpy as jnp
from jax.experimental import pallas as pl
from jax.experimental.pallas import tpu as pltpu

_N, _K, _D, _STAGES = 2048, 512, 64, 4
_NB = 256
_KB = 128


def _encoder_kernel(x_ref, cb_ref, bias_ref, enc_ref, cur_ref, loss_ref):
    x = x_ref[...]
    current = jnp.zeros_like(x)
    enc_cols = []
    for i in range(_STAGES):
        cur_i = current

        def chunk_body(c, carry, i=i, cur_i=cur_i):
            run_min, run_idx, run_vec = carry
            cb_c = cb_ref[i, pl.ds(c * _KB, _KB), :]
            cand = cur_i[:, None, :] + cb_c[None, :, :]
            if i == 0:
                cand = cand + bias_ref[pl.ds(c * _KB, _KB), :][None, :, :]
            diff = cand - x[:, None, :]
            losses = jnp.sum(diff * diff, axis=-1)
            loss_ref[:, pl.ds(i * _K + c * _KB, _KB)] = losses
            cmin = jnp.min(losses, axis=1, keepdims=True)
            iota = jax.lax.broadcasted_iota(jnp.int32, (_NB, _KB), 1)
            lidx = jnp.min(jnp.where(losses == cmin, iota, _K),
                           axis=1, keepdims=True)
            iota3 = jax.lax.broadcasted_iota(jnp.int32, (_NB, _KB, _D), 1)
            sel = iota3 == lidx[:, :, None]
            vec = jnp.sum(jnp.where(sel, cand, 0.0), axis=1)
            upd = cmin < run_min
            run_min = jnp.where(upd, cmin, run_min)
            run_idx = jnp.where(upd, lidx + c * _KB, run_idx)
            run_vec = jnp.where(upd, vec, run_vec)
            return run_min, run_idx, run_vec

        run_min, run_idx, run_vec = jax.lax.fori_loop(
            0, _K // _KB, chunk_body,
            (jnp.full((_NB, 1), jnp.inf, jnp.float32),
             jnp.zeros((_NB, 1), jnp.int32),
             jnp.zeros((_NB, _D), jnp.float32)))
        current = run_vec
        enc_cols.append(run_idx)
    enc_ref[...] = jnp.concatenate(enc_cols, axis=1)
    cur_ref[...] = current


def kernel(inputs, codebook, bias):
    enc, cur, losses = pl.pallas_call(
        _encoder_kernel,
        grid=(_N // _NB,),
        in_specs=[
            pl.BlockSpec((_NB, _D), lambda m: (m, 0)),
            pl.BlockSpec((_STAGES, _K, _D), lambda m: (0, 0, 0)),
            pl.BlockSpec((_K, _D), lambda m: (0, 0)),
        ],
        out_specs=[
            pl.BlockSpec((_NB, _STAGES), lambda m: (m, 0)),
            pl.BlockSpec((_NB, _D), lambda m: (m, 0)),
            pl.BlockSpec((_NB, _STAGES * _K), lambda m: (m, 0)),
        ],
        out_shape=[
            jax.ShapeDtypeStruct((_N, _STAGES), jnp.int32),
            jax.ShapeDtypeStruct((_N, _D), jnp.float32),
            jax.ShapeDtypeStruct((_N, _STAGES * _K), jnp.float32),
        ],
        compiler_params=pltpu.CompilerParams(
            dimension_semantics=("parallel",)),
    )(inputs, codebook, bias)
    return enc, cur, losses.reshape(_N, _STAGES, _K)


# speedup vs baseline: 4.3075x; 4.3075x over previous
import jax
import jax.numpy as jnp
from jax.experimental import pallas as pl
from jax.experimental.pallas import tpu as pltpu

_N, _K, _D, _STAGES = 2048, 512, 64, 4
_NB = 128
_KB = 128


def _encoder_kernel(x_ref, effT_ref, hi_ref, lo_ref,
                    enc_ref, cur_ref, loss_ref):
    x2 = x_ref[...]
    current = jnp.zeros_like(x2)
    iota_k = jax.lax.broadcasted_iota(jnp.int32, (_NB, _K), 1)
    enc_cols = []
    x3 = x2[:, :, None]
    for i in range(_STAGES):
        cur3 = current[:, :, None]

        def chunk_body(c, carry, i=i, cur3=cur3):
            run_min, run_idx = carry
            cbT_c = effT_ref[i, :, pl.ds(c * _KB, _KB)]
            cand = cur3 + cbT_c[None, :, :]
            diff = cand - x3
            sq = diff * diff
            losses = jnp.sum(sq, axis=1)
            loss_ref[:, pl.ds(i * _K + c * _KB, _KB)] = losses
            cmin = jnp.min(losses, axis=1, keepdims=True)
            iota = jax.lax.broadcasted_iota(jnp.int32, (_NB, _KB), 1)
            lidx = jnp.min(jnp.where(losses == cmin, iota, _K),
                           axis=1, keepdims=True) + c * _KB
            upd = cmin < run_min
            run_min = jnp.where(upd, cmin, run_min)
            run_idx = jnp.where(upd, lidx, run_idx)
            return run_min, run_idx

        _, run_idx = jax.lax.fori_loop(
            0, _K // _KB, chunk_body,
            (jnp.full((_NB, 1), jnp.inf, jnp.float32),
             jnp.zeros((_NB, 1), jnp.int32)))

        onehot = jnp.where(iota_k == run_idx, 1.0, 0.0)
        gh = jnp.dot(onehot, hi_ref[i], preferred_element_type=jnp.float32,
                     precision=jax.lax.Precision.HIGHEST)
        gl = jnp.dot(onehot, lo_ref[i], preferred_element_type=jnp.float32,
                     precision=jax.lax.Precision.HIGHEST)
        bits = (gh.astype(jnp.int32) << 16) | gl.astype(jnp.int32)
        csel = jax.lax.bitcast_convert_type(bits, jnp.float32)
        current = current + csel
        enc_cols.append(run_idx)
    enc_ref[...] = jnp.concatenate(enc_cols, axis=1)
    cur_ref[...] = current


def kernel(inputs, codebook, bias):
    eff = jnp.concatenate([(codebook[0] + bias)[None], codebook[1:]], axis=0)
    effT = jnp.swapaxes(eff, 1, 2)
    bits = jax.lax.bitcast_convert_type(eff, jnp.uint32)
    hi = (bits >> 16).astype(jnp.float32)
    lo = (bits & jnp.uint32(0xFFFF)).astype(jnp.float32)
    enc, cur, losses = pl.pallas_call(
        _encoder_kernel,
        grid=(_N // _NB,),
        in_specs=[
            pl.BlockSpec((_NB, _D), lambda m: (m, 0)),
            pl.BlockSpec((_STAGES, _D, _K), lambda m: (0, 0, 0)),
            pl.BlockSpec((_STAGES, _K, _D), lambda m: (0, 0, 0)),
            pl.BlockSpec((_STAGES, _K, _D), lambda m: (0, 0, 0)),
        ],
        out_specs=[
            pl.BlockSpec((_NB, _STAGES), lambda m: (m, 0)),
            pl.BlockSpec((_NB, _D), lambda m: (m, 0)),
            pl.BlockSpec((_NB, _STAGES * _K), lambda m: (m, 0)),
        ],
        out_shape=[
            jax.ShapeDtypeStruct((_N, _STAGES), jnp.int32),
            jax.ShapeDtypeStruct((_N, _D), jnp.float32),
            jax.ShapeDtypeStruct((_N, _STAGES * _K), jnp.float32),
        ],
        compiler_params=pltpu.CompilerParams(
            dimension_semantics=("parallel",)),
    )(inputs, effT, hi, lo)
    return enc, cur, losses.reshape(_N, _STAGES, _K)


# speedup vs baseline: 7.0409x; 1.6345x over previous
import jax
import jax.numpy as jnp
from jax.experimental import pallas as pl
from jax.experimental.pallas import tpu as pltpu

_N, _K, _D, _STAGES = 2048, 512, 64, 4
_NB = 256


def _exact_gather(onehot, hi, lo):
    gh = jnp.dot(onehot, hi, preferred_element_type=jnp.float32,
                 precision=jax.lax.Precision.HIGHEST)
    gl = jnp.dot(onehot, lo, preferred_element_type=jnp.float32,
                 precision=jax.lax.Precision.HIGHEST)
    bits = (gh.astype(jnp.int32) << 16) | gl.astype(jnp.int32)
    return jax.lax.bitcast_convert_type(bits, jnp.float32)


def _exact_loss(current, csel, x2):
    cand = current + csel
    diff = cand - x2
    sq = diff * diff
    v = sq[:, :, None]
    acc = v[:, 0:8]
    for j in range(1, 8):
        acc = acc + v[:, 8 * j:8 * (j + 1)]
    w = 4
    while w >= 1:
        acc = acc[:, :w] + acc[:, w:2 * w]
        w //= 2
    return acc[:, 0, :]


def _encoder_kernel(x_ref, effT_ref, hi_ref, lo_ref,
                    enc_ref, cur_ref, loss_ref):
    x2 = x_ref[...]
    current = jnp.zeros_like(x2)
    iota_k = jax.lax.broadcasted_iota(jnp.int32, (_NB, _K), 1)
    enc_cols = []
    for i in range(_STAGES):
        effT = effT_ref[i]
        r2 = x2 - current
        g = jnp.dot(r2, effT, preferred_element_type=jnp.float32,
                    precision=jax.lax.Precision.HIGHEST)
        nc = jnp.sum(effT * effT, axis=0, keepdims=True)
        q = jnp.sum(r2 * r2, axis=1, keepdims=True)
        mm = (q - 2.0 * g) + nc
        loss_ref[:, pl.ds(i * _K, _K)] = mm
        m1 = jnp.min(mm, axis=1, keepdims=True)
        i1 = jnp.min(jnp.where(mm == m1, iota_k, _K),
                     axis=1, keepdims=True)
        mmm = jnp.where(iota_k == i1, jnp.inf, mm)
        m2 = jnp.min(mmm, axis=1, keepdims=True)
        i2 = jnp.min(jnp.where(mmm == m2, iota_k, _K),
                     axis=1, keepdims=True)
        c1 = _exact_gather(jnp.where(iota_k == i1, 1.0, 0.0),
                           hi_ref[i], lo_ref[i])
        c2 = _exact_gather(jnp.where(iota_k == i2, 1.0, 0.0),
                           hi_ref[i], lo_ref[i])
        e1 = _exact_loss(current, c1, x2)
        e2 = _exact_loss(current, c2, x2)
        pick1 = (e1 < e2) | ((e1 == e2) & (i1 < i2))
        idxw = jnp.where(pick1, i1, i2)
        cselw = jnp.where(pick1, c1, c2)
        current = current + cselw
        enc_cols.append(idxw)
    enc_ref[...] = jnp.concatenate(enc_cols, axis=1)
    cur_ref[...] = current


def kernel(inputs, codebook, bias):
    eff = jnp.concatenate([(codebook[0] + bias)[None], codebook[1:]], axis=0)
    effT = jnp.swapaxes(eff, 1, 2)
    bits = jax.lax.bitcast_convert_type(eff, jnp.uint32)
    hi = (bits >> 16).astype(jnp.float32)
    lo = (bits & jnp.uint32(0xFFFF)).astype(jnp.float32)
    enc, cur, losses = pl.pallas_call(
        _encoder_kernel,
        grid=(_N // _NB,),
        in_specs=[
            pl.BlockSpec((_NB, _D), lambda m: (m, 0)),
            pl.BlockSpec((_STAGES, _D, _K), lambda m: (0, 0, 0)),
            pl.BlockSpec((_STAGES, _K, _D), lambda m: (0, 0, 0)),
            pl.BlockSpec((_STAGES, _K, _D), lambda m: (0, 0, 0)),
        ],
        out_specs=[
            pl.BlockSpec((_NB, _STAGES), lambda m: (m, 0)),
            pl.BlockSpec((_NB, _D), lambda m: (m, 0)),
            pl.BlockSpec((_NB, _STAGES * _K), lambda m: (m, 0)),
        ],
        out_shape=[
            jax.ShapeDtypeStruct((_N, _STAGES), jnp.int32),
            jax.ShapeDtypeStruct((_N, _D), jnp.float32),
            jax.ShapeDtypeStruct((_N, _STAGES * _K), jnp.float32),
        ],
        compiler_params=pltpu.CompilerParams(
            dimension_semantics=("parallel",)),
    )(inputs, effT, hi, lo)
    return enc, cur, losses.reshape(_N, _STAGES, _K)


# speedup vs baseline: 10.6920x; 1.5186x over previous
import jax
import jax.numpy as jnp
from jax.experimental import pallas as pl
from jax.experimental.pallas import tpu as pltpu

_N, _K, _D, _STAGES = 2048, 512, 64, 4
_NB = 256


def _exact_gather(onehot, hi, lo):
    gh = jnp.dot(onehot, hi, preferred_element_type=jnp.float32,
                 precision=jax.lax.Precision.HIGHEST)
    gl = jnp.dot(onehot, lo, preferred_element_type=jnp.float32,
                 precision=jax.lax.Precision.HIGHEST)
    bits = (gh.astype(jnp.int32) << 16) | gl.astype(jnp.int32)
    return jax.lax.bitcast_convert_type(bits, jnp.float32)


def _exact_loss(current, csel, x2):
    cand = current + csel
    diff = cand - x2
    sq = diff * diff
    acc = sq
    for j in range(1, 8):
        acc = acc + pltpu.roll(sq, _D - 8 * j, 1)
    for w in (4, 2, 1):
        acc = acc + pltpu.roll(acc, _D - w, 1)
    return acc[:, 0:1]


def _encoder_kernel(x_ref, effT_ref, hi_ref, lo_ref, nc_ref,
                    enc_ref, cur_ref, loss_ref):
    x2 = x_ref[...]
    current = jnp.zeros_like(x2)
    iota_k = jax.lax.broadcasted_iota(jnp.int32, (_NB, _K), 1)
    enc_cols = []
    for i in range(_STAGES):
        effT = effT_ref[i]
        r2 = x2 - current
        g = jnp.dot(r2, effT, preferred_element_type=jnp.float32,
                    precision=jax.lax.Precision.HIGHEST)
        nc = nc_ref[pl.ds(i, 1), :]
        q = jnp.sum(r2 * r2, axis=1, keepdims=True)
        mm = (q - 2.0 * g) + nc
        loss_ref[:, pl.ds(i * _K, _K)] = mm
        m1 = jnp.min(mm, axis=1, keepdims=True)
        i1 = jnp.min(jnp.where(mm == m1, iota_k, _K),
                     axis=1, keepdims=True)
        mmm = jnp.where(iota_k == i1, jnp.inf, mm)
        m2 = jnp.min(mmm, axis=1, keepdims=True)
        i2 = jnp.min(jnp.where(mmm == m2, iota_k, _K),
                     axis=1, keepdims=True)
        c1 = _exact_gather(jnp.where(iota_k == i1, 1.0, 0.0),
                           hi_ref[i], lo_ref[i])
        c2 = _exact_gather(jnp.where(iota_k == i2, 1.0, 0.0),
                           hi_ref[i], lo_ref[i])
        e1 = _exact_loss(current, c1, x2)
        e2 = _exact_loss(current, c2, x2)
        pick1 = (e1 < e2) | ((e1 == e2) & (i1 < i2))
        idxw = jnp.where(pick1, i1, i2)
        cselw = jnp.where(pick1, c1, c2)
        current = current + cselw
        enc_cols.append(idxw)
    enc_ref[...] = jnp.concatenate(enc_cols, axis=1)
    cur_ref[...] = current


def kernel(inputs, codebook, bias):
    eff = jnp.concatenate([(codebook[0] + bias)[None], codebook[1:]], axis=0)
    effT = jnp.swapaxes(eff, 1, 2)
    bits = jax.lax.bitcast_convert_type(eff, jnp.uint32)
    hi = (bits >> 16).astype(jnp.float32)
    lo = (bits & jnp.uint32(0xFFFF)).astype(jnp.float32)
    nc = jnp.sum(eff * eff, axis=2)
    enc, cur, losses = pl.pallas_call(
        _encoder_kernel,
        grid=(_N // _NB,),
        in_specs=[
            pl.BlockSpec((_NB, _D), lambda m: (m, 0)),
            pl.BlockSpec((_STAGES, _D, _K), lambda m: (0, 0, 0)),
            pl.BlockSpec((_STAGES, _K, _D), lambda m: (0, 0, 0)),
            pl.BlockSpec((_STAGES, _K, _D), lambda m: (0, 0, 0)),
            pl.BlockSpec((_STAGES, _K), lambda m: (0, 0)),
        ],
        out_specs=[
            pl.BlockSpec((_NB, _STAGES), lambda m: (m, 0)),
            pl.BlockSpec((_NB, _D), lambda m: (m, 0)),
            pl.BlockSpec((_NB, _STAGES * _K), lambda m: (m, 0)),
        ],
        out_shape=[
            jax.ShapeDtypeStruct((_N, _STAGES), jnp.int32),
            jax.ShapeDtypeStruct((_N, _D), jnp.float32),
            jax.ShapeDtypeStruct((_N, _STAGES * _K), jnp.float32),
        ],
        compiler_params=pltpu.CompilerParams(
            dimension_semantics=("parallel",)),
    )(inputs, effT, hi, lo, nc)
    return enc, cur, losses.reshape(_N, _STAGES, _K)


# speedup vs baseline: 11.9280x; 1.1156x over previous
import jax
import jax.numpy as jnp
from jax.experimental import pallas as pl
from jax.experimental.pallas import tpu as pltpu

_N, _K, _D, _STAGES = 2048, 512, 64, 4
_NB = 256


def _exact_gather(onehot, hi, lo):
    gh = jnp.dot(onehot, hi, preferred_element_type=jnp.float32,
                 precision=jax.lax.Precision.HIGHEST)
    gl = jnp.dot(onehot, lo, preferred_element_type=jnp.float32,
                 precision=jax.lax.Precision.HIGHEST)
    bits = (gh.astype(jnp.int32) << 16) | gl.astype(jnp.int32)
    return jax.lax.bitcast_convert_type(bits, jnp.float32)


def _exact_loss_pair(current, c1, c2, x2):
    cc = jnp.concatenate([c1, c2], axis=1)
    curx = jnp.concatenate([current, current], axis=1)
    xx = jnp.concatenate([x2, x2], axis=1)
    cand = curx + cc
    diff = cand - xx
    sq = diff * diff
    acc = sq
    for j in range(1, 8):
        acc = acc + pltpu.roll(sq, 2 * _D - 8 * j, 1)
    for w in (4, 2, 1):
        acc = acc + pltpu.roll(acc, 2 * _D - w, 1)
    dd = acc - pltpu.roll(acc, _D, 1)
    return dd[:, 0:1]


def _encoder_kernel(x_ref, effT_ref, hi_ref, lo_ref, nc_ref,
                    enc_ref, cur_ref, loss_ref):
    x2 = x_ref[...]
    current = jnp.zeros_like(x2)
    iota_k = jax.lax.broadcasted_iota(jnp.int32, (_NB, _K), 1)
    enc_cols = []
    for i in range(_STAGES):
        effT = effT_ref[i]
        r2 = x2 - current
        g = jnp.dot(r2, effT, preferred_element_type=jnp.float32,
                    precision=jax.lax.Precision.HIGHEST)
        nc = nc_ref[pl.ds(i, 1), :]
        q = jnp.sum(r2 * r2, axis=1, keepdims=True)
        mm = (q - 2.0 * g) + nc
        loss_ref[:, pl.ds(i * _K, _K)] = mm
        m1 = jnp.min(mm, axis=1, keepdims=True)
        i1 = jnp.min(jnp.where(mm == m1, iota_k, _K),
                     axis=1, keepdims=True)
        mmm = jnp.where(iota_k == i1, jnp.inf, mm)
        m2 = jnp.min(mmm, axis=1, keepdims=True)
        i2 = jnp.min(jnp.where(mmm == m2, iota_k, _K),
                     axis=1, keepdims=True)
        oh = jnp.concatenate([jnp.where(iota_k == i1, 1.0, 0.0),
                              jnp.where(iota_k == i2, 1.0, 0.0)], axis=0)
        c12 = _exact_gather(oh, hi_ref[i], lo_ref[i])
        c1, c2 = c12[:_NB], c12[_NB:]
        d12 = _exact_loss_pair(current, c1, c2, x2)
        pick1 = (d12 < 0.0) | ((d12 == 0.0) & (i1 < i2))
        idxw = jnp.where(pick1, i1, i2)
        cselw = jnp.where(pick1, c1, c2)
        current = current + cselw
        enc_cols.append(idxw)
    enc_ref[...] = jnp.concatenate(enc_cols, axis=1)
    cur_ref[...] = current


def kernel(inputs, codebook, bias):
    eff = jnp.concatenate([(codebook[0] + bias)[None], codebook[1:]], axis=0)
    effT = jnp.swapaxes(eff, 1, 2)
    bits = jax.lax.bitcast_convert_type(eff, jnp.uint32)
    hi = (bits >> 16).astype(jnp.float32)
    lo = (bits & jnp.uint32(0xFFFF)).astype(jnp.float32)
    nc = jnp.sum(eff * eff, axis=2)
    enc, cur, losses = pl.pallas_call(
        _encoder_kernel,
        grid=(_N // _NB,),
        in_specs=[
            pl.BlockSpec((_NB, _D), lambda m: (m, 0)),
            pl.BlockSpec((_STAGES, _D, _K), lambda m: (0, 0, 0)),
            pl.BlockSpec((_STAGES, _K, _D), lambda m: (0, 0, 0)),
            pl.BlockSpec((_STAGES, _K, _D), lambda m: (0, 0, 0)),
            pl.BlockSpec((_STAGES, _K), lambda m: (0, 0)),
        ],
        out_specs=[
            pl.BlockSpec((_NB, _STAGES), lambda m: (m, 0)),
            pl.BlockSpec((_NB, _D), lambda m: (m, 0)),
            pl.BlockSpec((_NB, _STAGES * _K), lambda m: (m, 0)),
        ],
        out_shape=[
            jax.ShapeDtypeStruct((_N, _STAGES), jnp.int32),
            jax.ShapeDtypeStruct((_N, _D), jnp.float32),
            jax.ShapeDtypeStruct((_N, _STAGES * _K), jnp.float32),
        ],
        compiler_params=pltpu.CompilerParams(
            dimension_semantics=("parallel",)),
    )(inputs, effT, hi, lo, nc)
    return enc, cur, losses.reshape(_N, _STAGES, _K)
